# Initial kernel scaffold; baseline (speedup 1.0000x reference)
import jax
import jax.numpy as jnp
from jax.experimental import pallas as pl


def kernel(x, table):
    raise NotImplementedError("write your pallas kernel here")


# speedup vs baseline: 5.7511x; 5.7511x over previous
import jax
import jax.numpy as jnp
from jax.experimental import pallas as pl
from jax.experimental.pallas import tpu as pltpu

D_MODEL = 768
MAX_REL = 32
SEQ = 512
G_ROWS = 1024
TAB_PAD = 128
BI = 8


def _build_g_body(tab_ref, g_ref):
    flat = jax.lax.broadcasted_iota(jnp.int32, (8 * G_ROWS, 1), 0)
    s = flat // G_ROWS
    k = flat % G_ROWS
    idx = jnp.clip(511 - k - s, -MAX_REL, MAX_REL) + MAX_REL
    cols = jax.lax.broadcasted_iota(jnp.int32, (1, TAB_PAD), 1)
    oh = (idx == cols).astype(jnp.float32)
    g_ref[:] = jnp.dot(oh, tab_ref[:], preferred_element_type=jnp.float32
                       ).reshape(8, G_ROWS, D_MODEL)


def _add_body(x_ref, g_ref, o_ref):
    i0 = pl.program_id(0) * BI
    base = pl.multiple_of(504 - i0, 8)
    xv = x_ref[:]
    for r in range(BI):
        o_ref[r] = xv + g_ref[7 - r, pl.ds(base, SEQ), :]


def kernel(x, table):
    x2d = x.reshape(SEQ, D_MODEL)
    tab = jnp.zeros((TAB_PAD, D_MODEL), jnp.float32).at[: 2 * MAX_REL + 1].set(table)

    g2 = pl.pallas_call(
        _build_g_body,
        out_shape=jax.ShapeDtypeStruct((8, G_ROWS, D_MODEL), jnp.float32),
    )(tab)

    out = pl.pallas_call(
        _add_body,
        grid=(SEQ // BI,),
        in_specs=[
            pl.BlockSpec((SEQ, D_MODEL), lambda i: (0, 0)),
            pl.BlockSpec((8, G_ROWS, D_MODEL), lambda i: (0, 0, 0)),
        ],
        out_specs=pl.BlockSpec((BI, SEQ, D_MODEL), lambda i: (i, 0, 0)),
        out_shape=jax.ShapeDtypeStruct((SEQ, SEQ, D_MODEL), jnp.float32),
        compiler_params=pltpu.CompilerParams(
            dimension_semantics=("arbitrary",),
        ),
    )(x2d, g2)
    return out
